# baseline (speedup 1.0000x reference)
import jax
import jax.numpy as jnp
from jax.experimental import pallas as pl
from jax.experimental.pallas import tpu as pltpu

_B, _N, _K = 4, 1024, 16
_R, _A = 5, 8
_TR = 0.0295
_D0, _D1 = 64, 96
_NC = 40
_RA = _R * _A


def _tpl_rows():
    radii = _TR * (jnp.arange(1, _R + 1, dtype=jnp.float32) / _R)
    ang = 2.0 * jnp.pi * jnp.arange(_A, dtype=jnp.float32) / _A
    t = jnp.stack([radii[:, None] * jnp.cos(ang)[None, :],
                   radii[:, None] * jnp.sin(ang)[None, :]], axis=-1)
    return t.reshape(_RA, 2).T


def _roll_weights(W, d_major):
    Wr = jnp.stack([jnp.roll(W, rot, axis=2) for rot in range(_A)], axis=0)
    if d_major:
        Wp = Wr.transpose(4, 2, 3, 0, 1)
    else:
        Wp = Wr.transpose(2, 3, 4, 0, 1)
    T = W.shape[0]
    d = W.shape[3]
    return Wp.reshape(_RA * d, _A * T)


def _body(x_ref, xt_ref, tpl_ref, w0_ref, b0_ref, w1_ref, b1_ref,
          wd_ref, bd_ref, out_ref, x1_ref):
    f32 = jnp.float32
    x = x_ref[0]
    xt = xt_ref[0]
    iota = jax.lax.broadcasted_iota(jnp.int32, (_N, _N), 1)

    d2 = jnp.zeros((_N, _N), f32)
    for c in range(3):
        diff = x[:, c:c + 1] - xt[c:c + 1, :]
        d2 = d2 + diff * diff

    tplx = tpl_ref[0:1, :]
    tply = tpl_ref[1:2, :]
    cx = x[:, 0:1]
    cy = x[:, 1:2]

    work = d2
    idx_list = []
    dist_list = []
    nx_list = []
    ny_list = []
    nz_list = []
    for k in range(_K):
        idx = jnp.argmin(work, axis=1, keepdims=True)
        sel = iota == idx
        work = jnp.where(sel, jnp.inf, work)
        idx_list.append(idx)
        nco = jax.lax.dot_general(sel.astype(f32), x,
                                  (((1,), (0,)), ((), ())),
                                  preferred_element_type=f32)
        nx = nco[:, 0:1]
        ny = nco[:, 1:2]
        nz = nco[:, 2:3]
        dx = (nx - cx) - tplx
        dy = (ny - cy) - tply
        dist_list.append(jnp.sqrt(dx * dx + dy * dy + 1e-12))
        nx_list.append(nx)
        ny_list.append(ny)
        nz_list.append(nz)

    masked = list(dist_list)
    sel3 = []
    wvals = []
    for j in range(3):
        mv = masked[0]
        for k in range(1, _K):
            mv = jnp.minimum(mv, masked[k])
        picked = jnp.zeros((_N, _RA), jnp.bool_)
        sj = []
        new_masked = []
        for k in range(_K):
            hitk = jnp.logical_and(jnp.logical_not(picked), masked[k] == mv)
            picked = jnp.logical_or(picked, hitk)
            sj.append(hitk)
            new_masked.append(jnp.where(hitk, jnp.inf, masked[k]))
        masked = new_masked
        sel3.append(sj)
        wvals.append(1.0 / (mv + 1e-8))
    wsum = wvals[0] + wvals[1] + wvals[2]
    wn = [wv / wsum for wv in wvals]
    wk = []
    for k in range(_K):
        acc = jnp.zeros((_N, _RA), f32)
        for j in range(3):
            acc = acc + jnp.where(sel3[j][k], wn[j], 0.0)
        wk.append(acc)

    i0x = jnp.zeros((_N, _RA), f32)
    i0y = jnp.zeros((_N, _RA), f32)
    i0z = jnp.zeros((_N, _RA), f32)
    for k in range(_K):
        i0x = i0x + wk[k] * nx_list[k]
        i0y = i0y + wk[k] * ny_list[k]
        i0z = i0z + wk[k] * nz_list[k]
    X0 = jnp.concatenate([i0x, i0y, i0z], axis=1)
    Y0 = jnp.dot(X0, w0_ref[...], preferred_element_type=f32)
    b0 = b0_ref[0:1, :]
    emb = None
    for rot in range(_A):
        yr = jnp.maximum(Y0[:, rot * _D0:(rot + 1) * _D0] + b0, 0.0)
        emb = yr if emb is None else jnp.maximum(emb, yr)

    nf_list = []
    for k in range(_K):
        sel = (iota == idx_list[k]).astype(f32)
        nf_list.append(jax.lax.dot_general(sel, emb,
                                           (((1,), (0,)), ((), ())),
                                           preferred_element_type=f32))
    Y1 = jnp.zeros((_N, _A * _D1), f32)
    half = _RA // 2
    for h in range(2):
        for i in range(half):
            ra = h * half + i
            acc = jnp.zeros((_N, _D0), f32)
            for k in range(_K):
                acc = acc + wk[k][:, ra:ra + 1] * nf_list[k]
            x1_ref[:, i * _D0:(i + 1) * _D0] = acc
        Y1 = Y1 + jnp.dot(
            x1_ref[...],
            w1_ref[h * half * _D0:(h + 1) * half * _D0, :],
            preferred_element_type=f32)
    b1 = b1_ref[0:1, :]
    emb1 = None
    for rot in range(_A):
        yr = jnp.maximum(Y1[:, rot * _D1:(rot + 1) * _D1] + b1, 0.0)
        emb1 = yr if emb1 is None else jnp.maximum(emb1, yr)

    mu = jnp.mean(emb1, axis=0, keepdims=True)
    xm = emb1 - mu
    cov = jax.lax.dot_general(xm, xm, (((0,), (0,)), ((), ())),
                              preferred_element_type=f32) / _N
    logits = jnp.zeros((1, _NC), f32)
    for d in range(_D1):
        logits = logits + jnp.dot(cov[d:d + 1, :],
                                  wd_ref[d * _D1:(d + 1) * _D1, :],
                                  preferred_element_type=f32)
    out_ref[0] = logits + bd_ref[0:1, :]


def kernel(inputs, W0, b0, W1, b1, Wd, bd):
    w0all = _roll_weights(W0, d_major=True)
    w1all = _roll_weights(W1, d_major=False)
    tpl = _tpl_rows()
    xt = inputs.transpose(0, 2, 1)
    out = pl.pallas_call(
        _body,
        grid=(_B,),
        in_specs=[
            pl.BlockSpec((1, _N, 3), lambda b: (b, 0, 0)),
            pl.BlockSpec((1, 3, _N), lambda b: (b, 0, 0)),
            pl.BlockSpec((2, _RA), lambda b: (0, 0)),
            pl.BlockSpec((_RA * 3, _A * _D0), lambda b: (0, 0)),
            pl.BlockSpec((1, _D0), lambda b: (0, 0)),
            pl.BlockSpec((_RA * _D0, _A * _D1), lambda b: (0, 0)),
            pl.BlockSpec((1, _D1), lambda b: (0, 0)),
            pl.BlockSpec((_D1 * _D1, _NC), lambda b: (0, 0)),
            pl.BlockSpec((1, _NC), lambda b: (0, 0)),
        ],
        out_specs=pl.BlockSpec((1, 1, _NC), lambda b: (b, 0, 0)),
        out_shape=jax.ShapeDtypeStruct((_B, 1, _NC), jnp.float32),
        compiler_params=pltpu.CompilerParams(
            dimension_semantics=("parallel",)),
        scratch_shapes=[pltpu.VMEM((_N, _RA * _D0 // 2), jnp.float32)],
    )(inputs, xt, tpl, w0all, b0.reshape(1, _D0), w1all,
      b1.reshape(1, _D1), Wd, bd.reshape(1, _NC))
    return out.reshape(_B, _NC)


# speedup vs baseline: 1.1015x; 1.1015x over previous
import jax
import jax.numpy as jnp
from jax.experimental import pallas as pl
from jax.experimental.pallas import tpu as pltpu

_B, _N, _K = 4, 1024, 16
_R, _A = 5, 8
_TR = 0.0295
_D0, _D1 = 64, 96
_NC = 40
_RA = _R * _A


def _tpl_rows():
    radii = _TR * (jnp.arange(1, _R + 1, dtype=jnp.float32) / _R)
    ang = 2.0 * jnp.pi * jnp.arange(_A, dtype=jnp.float32) / _A
    t = jnp.stack([radii[:, None] * jnp.cos(ang)[None, :],
                   radii[:, None] * jnp.sin(ang)[None, :]], axis=-1)
    return t.reshape(_RA, 2).T


def _roll_weights(W, d_major):
    Wr = jnp.stack([jnp.roll(W, rot, axis=2) for rot in range(_A)], axis=0)
    if d_major:
        Wp = Wr.transpose(4, 2, 3, 0, 1)
    else:
        Wp = Wr.transpose(2, 3, 4, 0, 1)
    T = W.shape[0]
    d = W.shape[3]
    return Wp.reshape(_RA * d, _A * T)


def _body(x_ref, xt_ref, tpl_ref, w0_ref, b0_ref, w1_ref, b1_ref,
          wd_ref, bd_ref, out_ref):
    f32 = jnp.float32
    x = x_ref[0]
    xt = xt_ref[0]
    iota = jax.lax.broadcasted_iota(jnp.int32, (_N, _N), 1)

    d2 = jnp.zeros((_N, _N), f32)
    for c in range(3):
        diff = x[:, c:c + 1] - xt[c:c + 1, :]
        d2 = d2 + diff * diff

    tplx = tpl_ref[0:1, :]
    tply = tpl_ref[1:2, :]
    cx = x[:, 0:1]
    cy = x[:, 1:2]

    work = d2
    idx_list = []
    dist_list = []
    nx_list = []
    ny_list = []
    nz_list = []
    for k in range(_K):
        idx = jnp.argmin(work, axis=1, keepdims=True)
        sel = iota == idx
        work = jnp.where(sel, jnp.inf, work)
        idx_list.append(idx)
        nco = jax.lax.dot_general(sel.astype(f32), x,
                                  (((1,), (0,)), ((), ())),
                                  preferred_element_type=f32)
        nx = nco[:, 0:1]
        ny = nco[:, 1:2]
        nz = nco[:, 2:3]
        dx = (nx - cx) - tplx
        dy = (ny - cy) - tply
        dist_list.append(jnp.sqrt(dx * dx + dy * dy + 1e-12))
        nx_list.append(nx)
        ny_list.append(ny)
        nz_list.append(nz)

    masked = list(dist_list)
    sel3 = []
    wvals = []
    for j in range(3):
        mv = masked[0]
        for k in range(1, _K):
            mv = jnp.minimum(mv, masked[k])
        picked = jnp.zeros((_N, _RA), jnp.bool_)
        sj = []
        new_masked = []
        for k in range(_K):
            hitk = jnp.logical_and(jnp.logical_not(picked), masked[k] == mv)
            picked = jnp.logical_or(picked, hitk)
            sj.append(hitk)
            new_masked.append(jnp.where(hitk, jnp.inf, masked[k]))
        masked = new_masked
        sel3.append(sj)
        wvals.append(1.0 / (mv + 1e-8))
    wsum = wvals[0] + wvals[1] + wvals[2]
    wn = [wv / wsum for wv in wvals]
    wk = []
    for k in range(_K):
        acc = jnp.zeros((_N, _RA), f32)
        for j in range(3):
            acc = acc + jnp.where(sel3[j][k], wn[j], 0.0)
        wk.append(acc)

    i0x = jnp.zeros((_N, _RA), f32)
    i0y = jnp.zeros((_N, _RA), f32)
    i0z = jnp.zeros((_N, _RA), f32)
    for k in range(_K):
        i0x = i0x + wk[k] * nx_list[k]
        i0y = i0y + wk[k] * ny_list[k]
        i0z = i0z + wk[k] * nz_list[k]
    X0 = jnp.concatenate([i0x, i0y, i0z], axis=1)
    Y0 = jnp.dot(X0, w0_ref[...], preferred_element_type=f32)
    b0 = b0_ref[0:1, :]
    emb = None
    for rot in range(_A):
        yr = jnp.maximum(Y0[:, rot * _D0:(rot + 1) * _D0] + b0, 0.0)
        emb = yr if emb is None else jnp.maximum(emb, yr)

    nf_list = []
    for k in range(_K):
        sel = (iota == idx_list[k]).astype(f32)
        nf_list.append(jax.lax.dot_general(sel, emb,
                                           (((1,), (0,)), ((), ())),
                                           preferred_element_type=f32))
    Y1 = jnp.zeros((_N, _A * _D1), f32)
    for ra in range(_RA):
        acc = jnp.zeros((_N, _D0), f32)
        for k in range(_K):
            acc = acc + wk[k][:, ra:ra + 1] * nf_list[k]
        Y1 = Y1 + jnp.dot(acc, w1_ref[ra * _D0:(ra + 1) * _D0, :],
                          preferred_element_type=f32)
    b1 = b1_ref[0:1, :]
    emb1 = None
    for rot in range(_A):
        yr = jnp.maximum(Y1[:, rot * _D1:(rot + 1) * _D1] + b1, 0.0)
        emb1 = yr if emb1 is None else jnp.maximum(emb1, yr)

    mu = jnp.mean(emb1, axis=0, keepdims=True)
    xm = emb1 - mu
    cov = jax.lax.dot_general(xm, xm, (((0,), (0,)), ((), ())),
                              preferred_element_type=f32) / _N
    logits = jnp.zeros((1, _NC), f32)
    for d in range(_D1):
        logits = logits + jnp.dot(cov[d:d + 1, :],
                                  wd_ref[d * _D1:(d + 1) * _D1, :],
                                  preferred_element_type=f32)
    out_ref[0] = logits + bd_ref[0:1, :]


def kernel(inputs, W0, b0, W1, b1, Wd, bd):
    w0all = _roll_weights(W0, d_major=True)
    w1all = _roll_weights(W1, d_major=False)
    tpl = _tpl_rows()
    xt = inputs.transpose(0, 2, 1)
    out = pl.pallas_call(
        _body,
        grid=(_B,),
        in_specs=[
            pl.BlockSpec((1, _N, 3), lambda b: (b, 0, 0)),
            pl.BlockSpec((1, 3, _N), lambda b: (b, 0, 0)),
            pl.BlockSpec((2, _RA), lambda b: (0, 0)),
            pl.BlockSpec((_RA * 3, _A * _D0), lambda b: (0, 0)),
            pl.BlockSpec((1, _D0), lambda b: (0, 0)),
            pl.BlockSpec((_RA * _D0, _A * _D1), lambda b: (0, 0)),
            pl.BlockSpec((1, _D1), lambda b: (0, 0)),
            pl.BlockSpec((_D1 * _D1, _NC), lambda b: (0, 0)),
            pl.BlockSpec((1, _NC), lambda b: (0, 0)),
        ],
        out_specs=pl.BlockSpec((1, 1, _NC), lambda b: (b, 0, 0)),
        out_shape=jax.ShapeDtypeStruct((_B, 1, _NC), jnp.float32),
        compiler_params=pltpu.CompilerParams(
            dimension_semantics=("parallel",)),
    )(inputs, xt, tpl, w0all, b0.reshape(1, _D0), w1all,
      b1.reshape(1, _D1), Wd, bd.reshape(1, _NC))
    return out.reshape(_B, _NC)


# speedup vs baseline: 1.1084x; 1.0063x over previous
import jax
import jax.numpy as jnp
from jax.experimental import pallas as pl
from jax.experimental.pallas import tpu as pltpu

_B, _N, _K = 4, 1024, 16
_R, _A = 5, 8
_TR = 0.0295
_D0, _D1 = 64, 96
_NC = 40
_RA = _R * _A


def _tpl_rows():
    radii = _TR * (jnp.arange(1, _R + 1, dtype=jnp.float32) / _R)
    ang = 2.0 * jnp.pi * jnp.arange(_A, dtype=jnp.float32) / _A
    t = jnp.stack([radii[:, None] * jnp.cos(ang)[None, :],
                   radii[:, None] * jnp.sin(ang)[None, :]], axis=-1)
    return t.reshape(_RA, 2).T


def _roll_weights(W, d_major):
    Wr = jnp.stack([jnp.roll(W, rot, axis=2) for rot in range(_A)], axis=0)
    if d_major:
        Wp = Wr.transpose(4, 2, 3, 0, 1)
    else:
        Wp = Wr.transpose(2, 3, 4, 0, 1)
    T = W.shape[0]
    d = W.shape[3]
    return Wp.reshape(_RA * d, _A * T)


def _body(x_ref, xt_ref, tpl_ref, w0_ref, b0_ref, w1_ref, b1_ref,
          wd_ref, bd_ref, out_ref):
    f32 = jnp.float32
    x = x_ref[0]
    xt = xt_ref[0]
    iota = jax.lax.broadcasted_iota(jnp.int32, (_N, _N), 1)

    d2 = jnp.zeros((_N, _N), f32)
    for c in range(3):
        diff = x[:, c:c + 1] - xt[c:c + 1, :]
        d2 = d2 + diff * diff

    tplx = tpl_ref[0:1, :]
    tply = tpl_ref[1:2, :]
    cx = x[:, 0:1]
    cy = x[:, 1:2]

    iota0f = jax.lax.broadcasted_iota(jnp.int32, (_N, _N), 0).astype(f32)
    work = d2
    idx_list = []
    dsq_list = []
    nx_list = []
    ny_list = []
    nz_list = []
    for k in range(_K):
        mval = jnp.min(work, axis=1, keepdims=True)
        hit = work == mval
        idx = jnp.min(jnp.where(hit, iota, _N), axis=1, keepdims=True)
        sel = iota == idx
        work = jnp.where(sel, jnp.inf, work)
        idx_list.append(idx)
        selt = (iota0f == idx.astype(f32).T).astype(f32)
        nco = jnp.dot(xt, selt, preferred_element_type=f32).T
        nx = nco[:, 0:1]
        ny = nco[:, 1:2]
        nz = nco[:, 2:3]
        dx = (nx - cx) - tplx
        dy = (ny - cy) - tply
        dsq_list.append(jnp.sqrt(dx * dx + dy * dy + 1e-12))
        nx_list.append(nx)
        ny_list.append(ny)
        nz_list.append(nz)

    masked = list(dsq_list)
    sel3 = []
    wvals = []
    for j in range(3):
        mv = masked[0]
        for k in range(1, _K):
            mv = jnp.minimum(mv, masked[k])
        picked = jnp.zeros((_N, _RA), jnp.bool_)
        sj = []
        new_masked = []
        for k in range(_K):
            hitk = jnp.logical_and(jnp.logical_not(picked), masked[k] == mv)
            picked = jnp.logical_or(picked, hitk)
            sj.append(hitk)
            new_masked.append(jnp.where(hitk, jnp.inf, masked[k]))
        masked = new_masked
        sel3.append(sj)
        wvals.append(1.0 / (mv + 1e-8))
    wsum = wvals[0] + wvals[1] + wvals[2]
    wn = [wv / wsum for wv in wvals]
    wk = []
    for k in range(_K):
        acc = jnp.zeros((_N, _RA), f32)
        for j in range(3):
            acc = acc + jnp.where(sel3[j][k], wn[j], 0.0)
        wk.append(acc)

    i0x = jnp.zeros((_N, _RA), f32)
    i0y = jnp.zeros((_N, _RA), f32)
    i0z = jnp.zeros((_N, _RA), f32)
    for k in range(_K):
        i0x = i0x + wk[k] * nx_list[k]
        i0y = i0y + wk[k] * ny_list[k]
        i0z = i0z + wk[k] * nz_list[k]
    X0 = jnp.concatenate([i0x, i0y, i0z], axis=1)
    Y0 = jnp.dot(X0, w0_ref[...], preferred_element_type=f32)
    b0 = b0_ref[0:1, :]
    emb = None
    for rot in range(_A):
        yr = jnp.maximum(Y0[:, rot * _D0:(rot + 1) * _D0] + b0, 0.0)
        emb = yr if emb is None else jnp.maximum(emb, yr)

    nf_list = []
    for k in range(_K):
        sel = (iota == idx_list[k]).astype(f32)
        nf_list.append(jax.lax.dot_general(sel, emb,
                                           (((1,), (0,)), ((), ())),
                                           preferred_element_type=f32))
    Y1 = jnp.zeros((_N, _A * _D1), f32)
    for ra in range(_RA):
        acc = jnp.zeros((_N, _D0), f32)
        for k in range(_K):
            acc = acc + wk[k][:, ra:ra + 1] * nf_list[k]
        Y1 = Y1 + jnp.dot(acc, w1_ref[ra * _D0:(ra + 1) * _D0, :],
                          preferred_element_type=f32)
    b1 = b1_ref[0:1, :]
    emb1 = None
    for rot in range(_A):
        yr = jnp.maximum(Y1[:, rot * _D1:(rot + 1) * _D1] + b1, 0.0)
        emb1 = yr if emb1 is None else jnp.maximum(emb1, yr)

    mu = jnp.mean(emb1, axis=0, keepdims=True)
    xm = emb1 - mu
    cov = jax.lax.dot_general(xm, xm, (((0,), (0,)), ((), ())),
                              preferred_element_type=f32) / _N
    logits = jnp.zeros((1, _NC), f32)
    for d in range(_D1):
        logits = logits + jnp.dot(cov[d:d + 1, :],
                                  wd_ref[d * _D1:(d + 1) * _D1, :],
                                  preferred_element_type=f32)
    out_ref[0] = logits + bd_ref[0:1, :]


def kernel(inputs, W0, b0, W1, b1, Wd, bd):
    w0all = _roll_weights(W0, d_major=True)
    w1all = _roll_weights(W1, d_major=False)
    tpl = _tpl_rows()
    xt = inputs.transpose(0, 2, 1)
    out = pl.pallas_call(
        _body,
        grid=(_B,),
        in_specs=[
            pl.BlockSpec((1, _N, 3), lambda b: (b, 0, 0)),
            pl.BlockSpec((1, 3, _N), lambda b: (b, 0, 0)),
            pl.BlockSpec((2, _RA), lambda b: (0, 0)),
            pl.BlockSpec((_RA * 3, _A * _D0), lambda b: (0, 0)),
            pl.BlockSpec((1, _D0), lambda b: (0, 0)),
            pl.BlockSpec((_RA * _D0, _A * _D1), lambda b: (0, 0)),
            pl.BlockSpec((1, _D1), lambda b: (0, 0)),
            pl.BlockSpec((_D1 * _D1, _NC), lambda b: (0, 0)),
            pl.BlockSpec((1, _NC), lambda b: (0, 0)),
        ],
        out_specs=pl.BlockSpec((1, 1, _NC), lambda b: (b, 0, 0)),
        out_shape=jax.ShapeDtypeStruct((_B, 1, _NC), jnp.float32),
        compiler_params=pltpu.CompilerParams(
            dimension_semantics=("parallel",)),
    )(inputs, xt, tpl, w0all, b0.reshape(1, _D0), w1all,
      b1.reshape(1, _D1), Wd, bd.reshape(1, _NC))
    return out.reshape(_B, _NC)


# speedup vs baseline: 1.1527x; 1.0399x over previous
import jax
import jax.numpy as jnp
from jax.experimental import pallas as pl
from jax.experimental.pallas import tpu as pltpu

_B, _N, _K = 4, 1024, 16
_R, _A = 5, 8
_TR = 0.0295
_D0, _D1 = 64, 96
_NC = 40
_RA = _R * _A


def _tpl_rows():
    radii = _TR * (jnp.arange(1, _R + 1, dtype=jnp.float32) / _R)
    ang = 2.0 * jnp.pi * jnp.arange(_A, dtype=jnp.float32) / _A
    t = jnp.stack([radii[:, None] * jnp.cos(ang)[None, :],
                   radii[:, None] * jnp.sin(ang)[None, :]], axis=-1)
    return t.reshape(_RA, 2).T


def _roll_weights(W, d_major):
    Wr = jnp.stack([jnp.roll(W, rot, axis=2) for rot in range(_A)], axis=0)
    if d_major:
        Wp = Wr.transpose(4, 2, 3, 0, 1)
    else:
        Wp = Wr.transpose(2, 3, 4, 0, 1)
    T = W.shape[0]
    d = W.shape[3]
    return Wp.reshape(_RA * d, _A * T)


def _body(x_ref, xt_ref, tpl_ref, w0_ref, b0_ref, w1_ref, b1_ref,
          wd_ref, bd_ref, out_ref):
    f32 = jnp.float32
    x = x_ref[0]
    xt = xt_ref[0]
    iota = jax.lax.broadcasted_iota(jnp.int32, (_N, _N), 1)

    d2 = jnp.zeros((_N, _N), f32)
    for c in range(3):
        diff = x[:, c:c + 1] - xt[c:c + 1, :]
        d2 = d2 + diff * diff

    tplx = tpl_ref[0:1, :]
    tply = tpl_ref[1:2, :]
    cx = x[:, 0:1]
    cy = x[:, 1:2]

    work = d2
    idx_list = []
    dist_list = []
    nx_list = []
    ny_list = []
    nz_list = []
    for k in range(_K):
        mval = jnp.min(work, axis=1, keepdims=True)
        hit = work == mval
        idx = jnp.min(jnp.where(hit, iota, _N), axis=1, keepdims=True)
        sel = iota == idx
        work = jnp.where(sel, jnp.inf, work)
        idx_list.append(idx)
        nco = jax.lax.dot_general(sel.astype(f32), x,
                                  (((1,), (0,)), ((), ())),
                                  preferred_element_type=f32)
        nx = nco[:, 0:1]
        ny = nco[:, 1:2]
        nz = nco[:, 2:3]
        dx = (nx - cx) - tplx
        dy = (ny - cy) - tply
        dist_list.append(jnp.sqrt(dx * dx + dy * dy + 1e-12))
        nx_list.append(nx)
        ny_list.append(ny)
        nz_list.append(nz)

    masked = list(dist_list)
    sel3 = []
    wvals = []
    for j in range(3):
        mv = masked[0]
        for k in range(1, _K):
            mv = jnp.minimum(mv, masked[k])
        picked = jnp.zeros((_N, _RA), jnp.bool_)
        sj = []
        new_masked = []
        for k in range(_K):
            hitk = jnp.logical_and(jnp.logical_not(picked), masked[k] == mv)
            picked = jnp.logical_or(picked, hitk)
            sj.append(hitk)
            new_masked.append(jnp.where(hitk, jnp.inf, masked[k]))
        masked = new_masked
        sel3.append(sj)
        wvals.append(1.0 / (mv + 1e-8))
    wsum = wvals[0] + wvals[1] + wvals[2]
    wn = [wv / wsum for wv in wvals]
    wk = []
    for k in range(_K):
        acc = jnp.zeros((_N, _RA), f32)
        for j in range(3):
            acc = acc + jnp.where(sel3[j][k], wn[j], 0.0)
        wk.append(acc)

    i0x = jnp.zeros((_N, _RA), f32)
    i0y = jnp.zeros((_N, _RA), f32)
    i0z = jnp.zeros((_N, _RA), f32)
    for k in range(_K):
        i0x = i0x + wk[k] * nx_list[k]
        i0y = i0y + wk[k] * ny_list[k]
        i0z = i0z + wk[k] * nz_list[k]
    X0 = jnp.concatenate([i0x, i0y, i0z], axis=1)
    Y0 = jnp.dot(X0, w0_ref[...], preferred_element_type=f32)
    b0 = b0_ref[0:1, :]
    emb = None
    for rot in range(_A):
        yr = jnp.maximum(Y0[:, rot * _D0:(rot + 1) * _D0] + b0, 0.0)
        emb = yr if emb is None else jnp.maximum(emb, yr)

    nf_list = []
    for k in range(_K):
        sel = (iota == idx_list[k]).astype(f32)
        nf_list.append(jax.lax.dot_general(sel, emb,
                                           (((1,), (0,)), ((), ())),
                                           preferred_element_type=f32))
    Y1 = jnp.zeros((_N, _A * _D1), f32)
    for ra in range(_RA):
        acc = jnp.zeros((_N, _D0), f32)
        for k in range(_K):
            acc = acc + wk[k][:, ra:ra + 1] * nf_list[k]
        Y1 = Y1 + jnp.dot(acc, w1_ref[ra * _D0:(ra + 1) * _D0, :],
                          preferred_element_type=f32)
    b1 = b1_ref[0:1, :]
    emb1 = None
    for rot in range(_A):
        yr = jnp.maximum(Y1[:, rot * _D1:(rot + 1) * _D1] + b1, 0.0)
        emb1 = yr if emb1 is None else jnp.maximum(emb1, yr)

    mu = jnp.mean(emb1, axis=0, keepdims=True)
    xm = emb1 - mu
    cov = jax.lax.dot_general(xm, xm, (((0,), (0,)), ((), ())),
                              preferred_element_type=f32) / _N
    logits = jnp.zeros((1, _NC), f32)
    for d in range(_D1):
        logits = logits + jnp.dot(cov[d:d + 1, :],
                                  wd_ref[d * _D1:(d + 1) * _D1, :],
                                  preferred_element_type=f32)
    out_ref[0] = logits + bd_ref[0:1, :]


def kernel(inputs, W0, b0, W1, b1, Wd, bd):
    w0all = _roll_weights(W0, d_major=True)
    w1all = _roll_weights(W1, d_major=False)
    tpl = _tpl_rows()
    xt = inputs.transpose(0, 2, 1)
    out = pl.pallas_call(
        _body,
        grid=(_B,),
        in_specs=[
            pl.BlockSpec((1, _N, 3), lambda b: (b, 0, 0)),
            pl.BlockSpec((1, 3, _N), lambda b: (b, 0, 0)),
            pl.BlockSpec((2, _RA), lambda b: (0, 0)),
            pl.BlockSpec((_RA * 3, _A * _D0), lambda b: (0, 0)),
            pl.BlockSpec((1, _D0), lambda b: (0, 0)),
            pl.BlockSpec((_RA * _D0, _A * _D1), lambda b: (0, 0)),
            pl.BlockSpec((1, _D1), lambda b: (0, 0)),
            pl.BlockSpec((_D1 * _D1, _NC), lambda b: (0, 0)),
            pl.BlockSpec((1, _NC), lambda b: (0, 0)),
        ],
        out_specs=pl.BlockSpec((1, 1, _NC), lambda b: (b, 0, 0)),
        out_shape=jax.ShapeDtypeStruct((_B, 1, _NC), jnp.float32),
        compiler_params=pltpu.CompilerParams(
            dimension_semantics=("parallel",)),
    )(inputs, xt, tpl, w0all, b0.reshape(1, _D0), w1all,
      b1.reshape(1, _D1), Wd, bd.reshape(1, _NC))
    return out.reshape(_B, _NC)
